# baseline (speedup 1.0000x reference)
import functools

import jax
import jax.numpy as jnp
from jax import lax
from jax.experimental import pallas as pl
from jax.experimental.pallas import tpu as pltpu
from jax.experimental.pallas import tpu_sc as plsc

_FDIM = 128
_BATCH = 16384
_D = _FDIM // 2
_HALF = _BATCH // 2

_NC = 2
_NS = 16
_NW = _NC * _NS
_BPW = _HALF // _NW
_IDXC = 128
_NCHUNK = _BPW // _IDXC

_INV2PI = 0.15915494309189535
_HI = 6.28125
_LO = 0.0019353071795864769
_S = (0.9994499856355539, -0.1658382205956817, 0.007998520295566539,
      -0.00014773645596885543)
_C = (0.9999710807349519, -0.49983754043485695, 0.04152226790055865,
      -0.001344099441249913, 1.9064759252396257e-05)


def _sc_gather(label_r, table):
    mesh = plsc.VectorSubcoreMesh(core_axis_name="c", subcore_axis_name="s")

    @functools.partial(
        pl.kernel,
        mesh=mesh,
        out_type=jax.ShapeDtypeStruct((_HALF, _FDIM), jnp.float32),
        scratch_types=[
            pltpu.VMEM((_NCHUNK, _IDXC), jnp.int32),
            pltpu.VMEM((_BPW, _FDIM), jnp.float32),
            [pltpu.SemaphoreType.DMA] * _NCHUNK,
            pltpu.SemaphoreType.DMA,
        ],
    )
    def k(label_hbm, table_hbm, out_hbm, idx_v, rows_v, gsems, osem):
        wid = lax.axis_index("s") * _NC + lax.axis_index("c")
        base = wid * _BPW
        pltpu.sync_copy(label_hbm.at[wid], idx_v)
        gathers = [
            pltpu.async_copy(
                table_hbm.at[idx_v.at[j]],
                rows_v.at[pl.ds(j * _IDXC, _IDXC)],
                gsems[j],
            )
            for j in range(_NCHUNK)
        ]
        out_copies = []
        for j in range(_NCHUNK):
            gathers[j].wait()
            out_copies.append(
                pltpu.async_copy(
                    rows_v.at[pl.ds(j * _IDXC, _IDXC)],
                    out_hbm.at[pl.ds(base + j * _IDXC, _IDXC)],
                    osem,
                )
            )
        for c in out_copies:
            c.wait()

    return k(label_r, table)


def _tc_body(t_ref, invd_ref, g_ref, o_ref):
    x = t_ref[...] * invd_ref[...]
    kf = jnp.floor(x * _INV2PI + 0.5)
    r = (x - kf * _HI) - kf * _LO
    z = r * r
    sp = _S[-1]
    for a in _S[-2::-1]:
        sp = sp * z + a
    sin_v = sp * r
    cp = _C[-1]
    for a in _C[-2::-1]:
        cp = cp * z + a
    emb = jnp.concatenate((sin_v, cp), axis=1)
    o_ref[...] = emb + g_ref[...]


def _tc_sin_add(t2, invd2, g):
    bb = 4096
    return pl.pallas_call(
        _tc_body,
        out_shape=jax.ShapeDtypeStruct((_HALF, _FDIM), jnp.float32),
        grid=(_HALF // bb,),
        in_specs=[
            pl.BlockSpec((bb, 1), lambda i: (i, 0)),
            pl.BlockSpec((1, _D), lambda i: (0, 0)),
            pl.BlockSpec((bb, _FDIM), lambda i: (i, 0)),
        ],
        out_specs=pl.BlockSpec((bb, _FDIM), lambda i: (i, 0)),
    )(t2, invd2, g)


def kernel(t, label, class_emb):
    label32 = label.astype(jnp.int32)
    denom = 10000.0 ** (jnp.arange(_D, dtype=jnp.float32) / (_D - 1))
    invd = (1.0 / denom).reshape(1, _D)
    outs = []
    for h in range(2):
        lbl = lax.dynamic_slice_in_dim(label32, h * _HALF, _HALF)
        th = lax.dynamic_slice_in_dim(t, h * _HALF, _HALF)
        g = _sc_gather(lbl.reshape(_NW, _NCHUNK, _IDXC), class_emb)
        outs.append(_tc_sin_add(th.reshape(_HALF, 1), invd, g))
    return jnp.concatenate(outs, axis=0)


# speedup vs baseline: 1.1817x; 1.1817x over previous
import functools

import jax
import jax.numpy as jnp
from jax import lax
from jax.experimental import pallas as pl
from jax.experimental.pallas import tpu as pltpu
from jax.experimental.pallas import tpu_sc as plsc

_FDIM = 128
_BATCH = 16384
_D = _FDIM // 2

_NC = 2
_NS = 16
_NW = _NC * _NS
_BPW = _BATCH // _NW
_IDXC = 128
_NCHUNK = _BPW // _IDXC

_MAGIC = 12582912.0
_INV2PI = 0.15915494309189535
_HI = 6.28125
_LO = 0.0019353071795864769
_S = (0.9994499856355539, -0.1658382205956817, 0.007998520295566539,
      -0.00014773645596885543)
_C = (0.9999710807349519, -0.49983754043485695, 0.04152226790055865,
      -0.001344099441249913, 1.9064759252396257e-05)


def _sc_gather(label_r, table):
    mesh = plsc.VectorSubcoreMesh(core_axis_name="c", subcore_axis_name="s")

    @functools.partial(
        pl.kernel,
        mesh=mesh,
        out_type=jax.ShapeDtypeStruct((_BATCH, _FDIM), jnp.float32),
        scratch_types=[
            pltpu.VMEM((_NCHUNK, _IDXC), jnp.int32),
            pltpu.VMEM((_BPW, _FDIM), jnp.float32),
            [pltpu.SemaphoreType.DMA] * _NCHUNK,
            pltpu.SemaphoreType.DMA,
        ],
    )
    def k(label_hbm, table_hbm, out_hbm, idx_v, rows_v, gsems, osem):
        wid = lax.axis_index("s") * _NC + lax.axis_index("c")
        base = wid * _BPW
        pltpu.sync_copy(label_hbm.at[wid], idx_v)
        gathers = [
            pltpu.async_copy(
                table_hbm.at[idx_v.at[j]],
                rows_v.at[pl.ds(j * _IDXC, _IDXC)],
                gsems[j],
            )
            for j in range(_NCHUNK)
        ]
        out_copies = []
        for j in range(_NCHUNK):
            gathers[j].wait()
            out_copies.append(
                pltpu.async_copy(
                    rows_v.at[pl.ds(j * _IDXC, _IDXC)],
                    out_hbm.at[pl.ds(base + j * _IDXC, _IDXC)],
                    osem,
                )
            )
        for c in out_copies:
            c.wait()

    return k(label_r, table)


def _tc_body(t_ref, invd_ref, g_ref, o_ref):
    x = t_ref[...] * invd_ref[...]
    kf = jnp.floor(x * _INV2PI + 0.5)
    r = (x - kf * _HI) - kf * _LO
    z = r * r
    sp = _S[-1]
    for a in _S[-2::-1]:
        sp = sp * z + a
    sin_v = sp * r
    cp = _C[-1]
    for a in _C[-2::-1]:
        cp = cp * z + a
    emb = jnp.concatenate((sin_v, cp), axis=1)
    o_ref[...] = emb + g_ref[...]


def _tc_sin_add(t2, invd2, g):
    bb = 4096
    return pl.pallas_call(
        _tc_body,
        out_shape=jax.ShapeDtypeStruct((_BATCH, _FDIM), jnp.float32),
        grid=(_BATCH // bb,),
        in_specs=[
            pl.BlockSpec((bb, 1), lambda i: (i, 0)),
            pl.BlockSpec((1, _D), lambda i: (0, 0)),
            pl.BlockSpec((bb, _FDIM), lambda i: (i, 0)),
        ],
        out_specs=pl.BlockSpec((bb, _FDIM), lambda i: (i, 0)),
    )(t2, invd2, g)


def kernel(t, label, class_emb):
    label_r = label.astype(jnp.int32).reshape(_NW, _NCHUNK, _IDXC)
    gathered = _sc_gather(label_r, class_emb)
    denom = 10000.0 ** (jnp.arange(_D, dtype=jnp.float32) / (_D - 1))
    invd = (1.0 / denom).reshape(1, _D)
    return _tc_sin_add(t.reshape(_BATCH, 1), invd, gathered)
